# baseline (speedup 1.0000x reference)
import functools

import jax
import jax.numpy as jnp
from jax import lax
from jax.experimental import pallas as pl
from jax.experimental.pallas import tpu as pltpu
from jax.experimental.pallas import tpu_sc as plsc

BATCH = 16384
HIST = 50
DIM = 32
B_TOTAL = BATCH * HIST

NUM_WORKERS = 32
CHUNK = BATCH // NUM_WORKERS
N_CHUNKS = HIST

_mesh = plsc.VectorSubcoreMesh(core_axis_name="c", subcore_axis_name="s")


@functools.partial(
    pl.kernel,
    mesh=_mesh,
    compiler_params=pltpu.CompilerParams(use_tc_tiling_on_sc=False),
    out_type=jax.ShapeDtypeStruct((HIST, BATCH, DIM), jnp.float32),
    scratch_types=[
        pltpu.VMEM((2, CHUNK), jnp.int32),
        pltpu.VMEM((2, CHUNK, DIM), jnp.float32),
        pltpu.SemaphoreType.DMA,
        pltpu.SemaphoreType.DMA,
        pltpu.SemaphoreType.DMA,
        pltpu.SemaphoreType.DMA,
    ],
)
def _gather_kernel(table_hbm, ids_hbm, out3_hbm, idx_v, rows_v, g0, g1, s0, s1):
    wid = lax.axis_index("s") * 2 + lax.axis_index("c")
    base = wid * CHUNK
    sem_g = (g0, g1)
    sem_s = (s0, s1)

    gathers = [None] * N_CHUNKS
    stores = [None] * N_CHUNKS
    for i in range(N_CHUNKS):
        b = i % 2
        off = i * BATCH + base
        if i >= 2:
            stores[i - 2].wait()
        pltpu.sync_copy(ids_hbm.at[pl.ds(off, CHUNK)], idx_v.at[b])
        gathers[i] = pltpu.async_copy(
            table_hbm.at[idx_v.at[b]], rows_v.at[b], sem_g[b]
        )
        if i >= 1:
            pb = (i - 1) % 2
            gathers[i - 1].wait()
            stores[i - 1] = pltpu.async_copy(
                rows_v.at[pb], out3_hbm.at[i - 1, pl.ds(base, CHUNK)], sem_s[pb]
            )
    last = N_CHUNKS - 1
    lb = last % 2
    gathers[last].wait()
    stores[last] = pltpu.async_copy(
        rows_v.at[lb], out3_hbm.at[last, pl.ds(base, CHUNK)], sem_s[lb]
    )
    stores[last - 1].wait()
    stores[last].wait()


def kernel(token_ids, embeddings):
    flat = token_ids.T.reshape(B_TOTAL).astype(jnp.int32)
    out = _gather_kernel(embeddings, flat)
    return out.transpose(1, 0, 2)


# speedup vs baseline: 1.0172x; 1.0172x over previous
import functools

import jax
import jax.numpy as jnp
from jax import lax
from jax.experimental import pallas as pl
from jax.experimental.pallas import tpu as pltpu
from jax.experimental.pallas import tpu_sc as plsc

BATCH = 16384
HIST = 50
DIM = 32
B_TOTAL = BATCH * HIST
NUM_EMB = 1000000

NUM_WORKERS = 32
PER_W = B_TOTAL // NUM_WORKERS
CHUNK = 1600
N_CHUNKS = PER_W // CHUNK

_mesh = plsc.VectorSubcoreMesh(core_axis_name="c", subcore_axis_name="s")


@functools.partial(
    pl.kernel,
    mesh=_mesh,
    compiler_params=pltpu.CompilerParams(use_tc_tiling_on_sc=False),
    out_type=jax.ShapeDtypeStruct((B_TOTAL, DIM), jnp.float32),
    scratch_types=[
        pltpu.VMEM((2, CHUNK), jnp.int32),
        pltpu.VMEM((2, CHUNK, DIM), jnp.float32),
        pltpu.SemaphoreType.DMA,
        pltpu.SemaphoreType.DMA,
        pltpu.SemaphoreType.DMA,
        pltpu.SemaphoreType.DMA,
    ],
)
def _gather_kernel(table_hbm, ids_hbm, out_hbm, idx_v, rows_v, g0, g1, s0, s1):
    wid = lax.axis_index("s") * 2 + lax.axis_index("c")
    base = wid * PER_W
    sem_g = (g0, g1)
    sem_s = (s0, s1)

    gathers = [None] * N_CHUNKS
    stores = [None] * N_CHUNKS
    for i in range(N_CHUNKS):
        b = i % 2
        off = base + i * CHUNK
        if i >= 2:
            stores[i - 2].wait()
        pltpu.sync_copy(ids_hbm.at[pl.ds(off, CHUNK)], idx_v.at[b])
        gathers[i] = pltpu.async_copy(
            table_hbm.at[idx_v.at[b]], rows_v.at[b], sem_g[b]
        )
        if i >= 1:
            pb = (i - 1) % 2
            gathers[i - 1].wait()
            poff = base + (i - 1) * CHUNK
            stores[i - 1] = pltpu.async_copy(
                rows_v.at[pb], out_hbm.at[pl.ds(poff, CHUNK)], sem_s[pb]
            )
    last = N_CHUNKS - 1
    lb = last % 2
    gathers[last].wait()
    stores[last] = pltpu.async_copy(
        rows_v.at[lb], out_hbm.at[pl.ds(base + last * CHUNK, CHUNK)], sem_s[lb]
    )
    stores[last - 1].wait()
    stores[last].wait()


def kernel(token_ids, embeddings):
    flat = token_ids.T.reshape(B_TOTAL).astype(jnp.int32)
    emb_c = lax.optimization_barrier(embeddings.reshape(250000, 128))
    out = _gather_kernel(emb_c.reshape(NUM_EMB, DIM), flat)
    return out.reshape(HIST, BATCH, DIM).transpose(1, 0, 2)


# speedup vs baseline: 1.0177x; 1.0005x over previous
import functools

import jax
import jax.numpy as jnp
from jax import lax
from jax.experimental import pallas as pl
from jax.experimental.pallas import tpu as pltpu
from jax.experimental.pallas import tpu_sc as plsc

BATCH = 16384
HIST = 50
DIM = 32
B_TOTAL = BATCH * HIST
NUM_EMB = 1000000

NUM_WORKERS = 32
PER_W = B_TOTAL // NUM_WORKERS
CHUNK = 1600
N_CHUNKS = PER_W // CHUNK

_mesh = plsc.VectorSubcoreMesh(core_axis_name="c", subcore_axis_name="s")


@functools.partial(
    pl.kernel,
    mesh=_mesh,
    compiler_params=pltpu.CompilerParams(use_tc_tiling_on_sc=False),
    out_type=jax.ShapeDtypeStruct((B_TOTAL, DIM), jnp.float32),
    scratch_types=[
        pltpu.VMEM((2, CHUNK), jnp.int32),
        pltpu.VMEM((2, CHUNK, DIM), jnp.float32),
        pltpu.SemaphoreType.DMA,
        pltpu.SemaphoreType.DMA,
        pltpu.SemaphoreType.DMA,
        pltpu.SemaphoreType.DMA,
    ],
)
def _gather_kernel(table_hbm, ids_hbm, out_hbm, idx_v, rows_v, g0, g1, s0, s1):
    wid = lax.axis_index("s") * 2 + lax.axis_index("c")
    base = wid * PER_W
    sem_g = (g0, g1)
    sem_s = (s0, s1)

    gathers = [None] * N_CHUNKS
    stores = [None] * N_CHUNKS
    for i in range(N_CHUNKS):
        b = i % 2
        off = base + i * CHUNK
        if i >= 2:
            stores[i - 2].wait()
        pltpu.sync_copy(ids_hbm.at[pl.ds(off, CHUNK)], idx_v.at[b])
        gathers[i] = pltpu.async_copy(
            table_hbm.at[idx_v.at[b]], rows_v.at[b], sem_g[b]
        )
        if i >= 1:
            pb = (i - 1) % 2
            gathers[i - 1].wait()
            poff = base + (i - 1) * CHUNK
            stores[i - 1] = pltpu.async_copy(
                rows_v.at[pb], out_hbm.at[pl.ds(poff, CHUNK)], sem_s[pb]
            )
    last = N_CHUNKS - 1
    lb = last % 2
    gathers[last].wait()
    stores[last] = pltpu.async_copy(
        rows_v.at[lb], out_hbm.at[pl.ds(base + last * CHUNK, CHUNK)], sem_s[lb]
    )
    stores[last - 1].wait()
    stores[last].wait()


def kernel(token_ids, embeddings):
    flat = token_ids.T.reshape(B_TOTAL).astype(jnp.int32)
    emb_c = lax.optimization_barrier(embeddings.reshape(NUM_EMB * DIM))
    out = _gather_kernel(emb_c.reshape(NUM_EMB, DIM), flat)
    return out.reshape(HIST, BATCH, DIM).transpose(1, 0, 2)


# speedup vs baseline: 1.2110x; 1.1899x over previous
import functools

import jax
import jax.numpy as jnp
from jax import lax
from jax.experimental import pallas as pl
from jax.experimental.pallas import tpu as pltpu
from jax.experimental.pallas import tpu_sc as plsc

BATCH = 16384
HIST = 50
DIM = 32
B_TOTAL = BATCH * HIST
NUM_EMB = 1000000

NUM_WORKERS = 32
PER_W = B_TOTAL // NUM_WORKERS
CHUNK = 1600
N_CHUNKS = PER_W // CHUNK

_mesh = plsc.VectorSubcoreMesh(core_axis_name="c", subcore_axis_name="s")


@functools.partial(
    pl.kernel,
    mesh=_mesh,
    compiler_params=pltpu.CompilerParams(use_tc_tiling_on_sc=False),
    out_type=jax.ShapeDtypeStruct((B_TOTAL, DIM), jnp.float32),
    scratch_types=[
        pltpu.VMEM((2, CHUNK), jnp.int32),
        pltpu.VMEM((2, CHUNK, DIM), jnp.float32),
        pltpu.SemaphoreType.DMA,
        pltpu.SemaphoreType.DMA,
        pltpu.SemaphoreType.DMA,
        pltpu.SemaphoreType.DMA,
    ],
)
def _gather_kernel(table_hbm, ids_hbm, out_hbm, idx_v, rows_v, g0, g1, s0, s1):
    wid = lax.axis_index("s") * 2 + lax.axis_index("c")
    base = wid * PER_W
    sem_g = (g0, g1)
    sem_s = (s0, s1)

    gathers = [None] * N_CHUNKS
    stores = [None] * N_CHUNKS
    for i in range(N_CHUNKS):
        b = i % 2
        off = base + i * CHUNK
        if i >= 2:
            stores[i - 2].wait()
        pltpu.sync_copy(ids_hbm.at[pl.ds(off, CHUNK)], idx_v.at[b])
        gathers[i] = pltpu.async_copy(
            table_hbm.at[idx_v.at[b]], rows_v.at[b], sem_g[b]
        )
        if i >= 1:
            pb = (i - 1) % 2
            gathers[i - 1].wait()
            poff = base + (i - 1) * CHUNK
            stores[i - 1] = pltpu.async_copy(
                rows_v.at[pb], out_hbm.at[pl.ds(poff, CHUNK)], sem_s[pb]
            )
    last = N_CHUNKS - 1
    lb = last % 2
    gathers[last].wait()
    stores[last] = pltpu.async_copy(
        rows_v.at[lb], out_hbm.at[pl.ds(base + last * CHUNK, CHUNK)], sem_s[lb]
    )
    stores[last - 1].wait()
    stores[last].wait()


def kernel(token_ids, embeddings):
    flat = token_ids.T.reshape(B_TOTAL).astype(jnp.int32)
    emb_c = lax.optimization_barrier(embeddings.reshape(NUM_EMB * DIM))
    out = _gather_kernel(emb_c.reshape(NUM_EMB, DIM), flat)
    out_c = lax.optimization_barrier(out.reshape(HIST, BATCH * DIM // 128, 128))
    return out_c.reshape(HIST, BATCH, DIM).transpose(1, 0, 2)
